# baseline (speedup 1.0000x reference)
import jax
import jax.numpy as jnp
from jax.experimental import pallas as pl


def _copy_body(x_ref, o_ref):
    o_ref[...] = x_ref[...]


def kernel(features):
    F, B, C, H, W = features.shape
    N = C * H * W
    LANES = 128
    rows = N // LANES
    x = jnp.reshape(features, (F, B, rows, LANES))

    out = pl.pallas_call(
        _copy_body,
        grid=(B, F),
        in_specs=[pl.BlockSpec((1, 1, rows, LANES), lambda b, i: (i, b, 0, 0))],
        out_specs=pl.BlockSpec((1, 1, rows, LANES), lambda b, i: (b, i, 0, 0)),
        out_shape=jax.ShapeDtypeStruct((B, F, rows, LANES), features.dtype),
    )(x)
    return jnp.reshape(out, (B, F * C * H, W))
